# baseline (speedup 1.0000x reference)
import functools

import jax
import jax.numpy as jnp
from jax import lax
from jax.experimental import pallas as pl
from jax.experimental.pallas import tpu as pltpu
from jax.experimental.pallas import tpu_sc as plsc

V = 1000
B = 1024
T = 50
N = B * T

_LANES = 16
_BW = 32
_HW = 16
_VT = V // 8
_NU = T * (_BW // _HW)


def _lse_body(tab_ref, out_ref):
    x = tab_ref[...]
    m = jnp.max(x, axis=1)
    out_ref[...] = m + jnp.log(jnp.sum(jnp.exp(x - m[:, None]), axis=1))


def _row_lse(table):
    return pl.pallas_call(
        _lse_body,
        out_shape=jax.ShapeDtypeStruct((V,), jnp.float32),
    )(table)


def _make_sc_kernel():
    info = plsc.get_sparse_core_info()
    nc, ns = info.num_cores, info.num_subcores
    nw = nc * ns

    mesh = plsc.VectorSubcoreMesh(core_axis_name="c", subcore_axis_name="s")

    @functools.partial(
        pl.kernel,
        mesh=mesh,
        compiler_params=pltpu.CompilerParams(
            needs_layout_passes=False, use_tc_tiling_on_sc=False
        ),
        out_type=[
            jax.ShapeDtypeStruct((T, _VT, 8, 8, 128), jnp.float32),
            jax.ShapeDtypeStruct((nw, _LANES), jnp.float32),
        ],
        scratch_types=[
            pltpu.VMEM((2, _HW), jnp.int32),
            pltpu.VMEM((_HW,), jnp.int32),
            pltpu.VMEM((V,), jnp.float32),
            pltpu.VMEM((2, _HW, V), jnp.float32),
            pltpu.VMEM((2, _VT, 8, _HW), jnp.float32),
            pltpu.VMEM((_LANES,), jnp.float32),
            pltpu.SemaphoreType.DMA((2,)),
            pltpu.SemaphoreType.DMA((2,)),
        ],
    )
    def sc_kernel(idxT_hbm, tgtT_hbm, table_hbm, lse_hbm,
                  out_hbm, part_hbm,
                  idxb, tgtb, lse_v, gbuf, tbuf, stage_v, gsem, osem):
        wid = lax.axis_index("s") * nc + lax.axis_index("c")
        b_lo = wid * _BW
        bt0 = b_lo // 128
        bi0 = b_lo % 128
        pltpu.sync_copy(lse_hbm, lse_v)

        lane = lax.iota(jnp.int32, _LANES)

        def start_gather(u, p):
            t = u // 2
            boff = b_lo + (u % 2) * _HW
            pltpu.sync_copy(idxT_hbm.at[t, pl.ds(boff, _HW)], idxb.at[p])
            pltpu.make_async_copy(
                table_hbm.at[idxb.at[p]], gbuf.at[p], gsem.at[p]
            ).start()

        def wait_gather(p):
            pltpu.make_async_copy(
                table_hbm.at[idxb.at[p]], gbuf.at[p], gsem.at[p]
            ).wait()

        def out_copy(u, p):
            t = u // 2
            bi = bi0 + (u % 2) * _HW
            return pltpu.make_async_copy(
                tbuf.at[p],
                out_hbm.at[t, pl.ds(0, _VT), bt0, pl.ds(0, 8), pl.ds(bi, _HW)],
                osem.at[p],
            )

        def loss_accum(u, p, acc):
            t = u // 2
            boff = b_lo + (u % 2) * _HW
            pltpu.sync_copy(tgtT_hbm.at[t, pl.ds(boff, _HW)], tgtb)
            idx_vals = idxb[p, pl.ds(0, _LANES)]
            tgt_vals = tgtb[pl.ds(0, _LANES)]
            lse_vals = plsc.load_gather(lse_v, [idx_vals])
            tl = plsc.load_gather(gbuf.at[p], [lane, tgt_vals])
            return acc + (lse_vals - tl)

        def transpose(p):
            def vt_step(vt, _):
                for vi in range(8):
                    v = vt * 8 + vi
                    vvec = jnp.full((_LANES,), v, jnp.int32)
                    vals = plsc.load_gather(gbuf.at[p], [lane, vvec])
                    tbuf[p, vt, vi, pl.ds(0, _LANES)] = vals
                return 0
            lax.fori_loop(0, _VT, vt_step, 0)

        def step(u, p, acc, *, do_wait_out, prefetch):
            wait_gather(p)
            acc = loss_accum(u, p, acc)
            if do_wait_out:
                out_copy(u - 2, p).wait()
            transpose(p)
            out_copy(u, p).start()
            if prefetch:
                start_gather(u + 2, p)
            return acc

        start_gather(0, 0)
        start_gather(1, 1)
        acc = jnp.zeros((_LANES,), jnp.float32)
        acc = step(0, 0, acc, do_wait_out=False, prefetch=True)
        acc = step(1, 1, acc, do_wait_out=False, prefetch=True)

        def main_body(uu, acc):
            u = 2 + uu * 2
            acc = step(u, 0, acc, do_wait_out=True, prefetch=True)
            acc = step(u + 1, 1, acc, do_wait_out=True, prefetch=True)
            return acc

        acc = lax.fori_loop(0, (_NU - 6) // 2, main_body, acc)
        acc = step(_NU - 4, 0, acc, do_wait_out=True, prefetch=True)
        acc = step(_NU - 3, 1, acc, do_wait_out=True, prefetch=True)
        acc = step(_NU - 2, 0, acc, do_wait_out=True, prefetch=False)
        acc = step(_NU - 1, 1, acc, do_wait_out=True, prefetch=False)
        out_copy(_NU - 2, 0).wait()
        out_copy(_NU - 1, 1).wait()

        stage_v[...] = acc
        pltpu.sync_copy(stage_v, part_hbm.at[wid])

    return sc_kernel


def kernel(idx, targets, token_embedding_table):
    idx_t = idx.astype(jnp.int32).T
    tgt_t = targets.astype(jnp.int32).T
    table = token_embedding_table.astype(jnp.float32)
    lse = _row_lse(table)
    out5, partials = _make_sc_kernel()(idx_t, tgt_t, table, lse)
    logits = out5.transpose(2, 4, 0, 1, 3).reshape(B, T, V)
    loss = jnp.sum(partials) / N
    return logits, loss


# speedup vs baseline: 1.1622x; 1.1622x over previous
import functools

import jax
import jax.numpy as jnp
from jax import lax
from jax.experimental import pallas as pl
from jax.experimental.pallas import tpu as pltpu
from jax.experimental.pallas import tpu_sc as plsc

V = 1000
B = 1024
T = 50
N = B * T

_LANES = 16
_BW = 32
_HW = 16
_VT = V // 8
_NU = T * (_BW // _HW)


def _lse_body(tab_ref, out_ref):
    x = tab_ref[...]
    m = jnp.max(x, axis=1)
    out_ref[...] = m + jnp.log(jnp.sum(jnp.exp(x - m[:, None]), axis=1))


def _row_lse(table):
    return pl.pallas_call(
        _lse_body,
        out_shape=jax.ShapeDtypeStruct((V,), jnp.float32),
    )(table)


def _make_sc_kernel():
    info = plsc.get_sparse_core_info()
    nc, ns = info.num_cores, info.num_subcores
    nw = nc * ns

    mesh = plsc.VectorSubcoreMesh(core_axis_name="c", subcore_axis_name="s")

    @functools.partial(
        pl.kernel,
        mesh=mesh,
        compiler_params=pltpu.CompilerParams(
            needs_layout_passes=False, use_tc_tiling_on_sc=False
        ),
        out_type=[
            jax.ShapeDtypeStruct((T, _VT, 8, 8, 128), jnp.float32),
            jax.ShapeDtypeStruct((nw, _LANES), jnp.float32),
        ],
        scratch_types=[
            pltpu.VMEM((T, _BW), jnp.int32),
            pltpu.VMEM((T, _BW), jnp.int32),
            pltpu.VMEM((V,), jnp.float32),
            pltpu.VMEM((2, _HW, V), jnp.float32),
            pltpu.VMEM((2, _VT, 8, _HW), jnp.float32),
            pltpu.VMEM((_LANES,), jnp.float32),
            pltpu.SemaphoreType.DMA((2,)),
            pltpu.SemaphoreType.DMA((2,)),
        ],
    )
    def sc_kernel(idxT_hbm, tgtT_hbm, table_hbm, lse_hbm,
                  out_hbm, part_hbm,
                  idxb, tgtb, lse_v, gbuf, tbuf, stage_v, gsem, osem):
        wid = lax.axis_index("s") * nc + lax.axis_index("c")
        b_lo = wid * _BW
        bt0 = b_lo // 128
        bi0 = b_lo % 128
        pltpu.sync_copy(lse_hbm, lse_v)
        pltpu.sync_copy(idxT_hbm.at[pl.ds(0, T), pl.ds(b_lo, _BW)], idxb)
        pltpu.sync_copy(tgtT_hbm.at[pl.ds(0, T), pl.ds(b_lo, _BW)], tgtb)

        lane = lax.iota(jnp.int32, _LANES)

        def start_gather(u, p):
            pltpu.make_async_copy(
                table_hbm.at[idxb.at[u // 2, pl.ds((u % 2) * _HW, _HW)]],
                gbuf.at[p], gsem.at[p],
            ).start()

        def wait_gather(u, p):
            pltpu.make_async_copy(
                table_hbm.at[idxb.at[u // 2, pl.ds((u % 2) * _HW, _HW)]],
                gbuf.at[p], gsem.at[p],
            ).wait()

        def out_copy(u, p):
            t = u // 2
            bi = bi0 + (u % 2) * _HW
            return pltpu.make_async_copy(
                tbuf.at[p],
                out_hbm.at[t, pl.ds(0, _VT), bt0, pl.ds(0, 8), pl.ds(bi, _HW)],
                osem.at[p],
            )

        def loss_accum(u, p, acc):
            t = u // 2
            boff = (u % 2) * _HW
            idx_vals = idxb[t, pl.ds(boff, _LANES)]
            tgt_vals = tgtb[t, pl.ds(boff, _LANES)]
            lse_vals = plsc.load_gather(lse_v, [idx_vals])
            tl = plsc.load_gather(gbuf.at[p], [lane, tgt_vals])
            return acc + (lse_vals - tl)

        def transpose(p):
            def vt_step(vt5, _):
                base = jnp.full((_LANES,), vt5 * 40, jnp.int32)
                for k in range(5):
                    for vi in range(8):
                        vvec = base + (k * 8 + vi)
                        vals = plsc.load_gather(gbuf.at[p], [lane, vvec])
                        tbuf[p, vt5 * 5 + k, vi, pl.ds(0, _LANES)] = vals
                return 0
            lax.fori_loop(0, _VT // 5, vt_step, 0)

        def step(u, p, acc, *, do_wait_out, prefetch):
            wait_gather(u, p)
            acc = loss_accum(u, p, acc)
            if do_wait_out:
                out_copy(u - 2, p).wait()
            transpose(p)
            out_copy(u, p).start()
            if prefetch:
                start_gather(u + 2, p)
            return acc

        start_gather(0, 0)
        start_gather(1, 1)
        acc = jnp.zeros((_LANES,), jnp.float32)
        acc = step(0, 0, acc, do_wait_out=False, prefetch=True)
        acc = step(1, 1, acc, do_wait_out=False, prefetch=True)

        def main_body(uu, acc):
            u = 2 + uu * 2
            acc = step(u, 0, acc, do_wait_out=True, prefetch=True)
            acc = step(u + 1, 1, acc, do_wait_out=True, prefetch=True)
            return acc

        acc = lax.fori_loop(0, (_NU - 6) // 2, main_body, acc)
        acc = step(_NU - 4, 0, acc, do_wait_out=True, prefetch=True)
        acc = step(_NU - 3, 1, acc, do_wait_out=True, prefetch=True)
        acc = step(_NU - 2, 0, acc, do_wait_out=True, prefetch=False)
        acc = step(_NU - 1, 1, acc, do_wait_out=True, prefetch=False)
        out_copy(_NU - 2, 0).wait()
        out_copy(_NU - 1, 1).wait()

        stage_v[...] = acc
        pltpu.sync_copy(stage_v, part_hbm.at[wid])

    return sc_kernel


def kernel(idx, targets, token_embedding_table):
    idx_t = idx.astype(jnp.int32).T
    tgt_t = targets.astype(jnp.int32).T
    table = token_embedding_table.astype(jnp.float32)
    lse = _row_lse(table)
    out5, partials = _make_sc_kernel()(idx_t, tgt_t, table, lse)
    logits = out5.transpose(2, 4, 0, 1, 3).reshape(B, T, V)
    loss = jnp.sum(partials) / N
    return logits, loss
